# baseline (speedup 1.0000x reference)
import jax
import jax.numpy as jnp
from jax.experimental import pallas as pl
from jax.experimental.pallas import tpu as pltpu

_ROWS = 4096
_FEATS = 26
_NUM_CLASSES = 1000
_BLOCK_ROWS = 128


def _onehot_block(indices_ref, depth_ref, values_ref, out_ref):
    idx = indices_ref[...]
    classes = jax.lax.broadcasted_iota(
        jnp.int32, (_BLOCK_ROWS * _FEATS, _NUM_CLASSES), 1
    )
    off = values_ref[0]
    on = values_ref[1]
    hit = (idx == classes) & (classes < depth_ref[0])
    out_ref[...] = jnp.where(hit, on, off)


def kernel(indices, depth, values):
    grid = _ROWS // _BLOCK_ROWS
    depth_arr = jnp.asarray(depth, dtype=jnp.int32).reshape(1)
    idx_col = indices.reshape(_ROWS * _FEATS, 1)
    out = pl.pallas_call(
        _onehot_block,
        grid=(grid,),
        in_specs=[
            pl.BlockSpec((_BLOCK_ROWS * _FEATS, 1), lambda i: (i, 0)),
            pl.BlockSpec(memory_space=pltpu.SMEM),
            pl.BlockSpec(memory_space=pltpu.SMEM),
        ],
        out_specs=pl.BlockSpec(
            (_BLOCK_ROWS * _FEATS, _NUM_CLASSES), lambda i: (i, 0)
        ),
        out_shape=jax.ShapeDtypeStruct(
            (_ROWS * _FEATS, _NUM_CLASSES), jnp.float32
        ),
    )(idx_col, depth_arr, values)
    return out.reshape(_ROWS, _FEATS, _NUM_CLASSES)


# speedup vs baseline: 1.3317x; 1.3317x over previous
import jax
import jax.numpy as jnp
from jax.experimental import pallas as pl
from jax.experimental.pallas import tpu as pltpu

_ROWS = 4096
_FEATS = 26
_NUM_CLASSES = 1000
_BLOCK_ROWS = 128


def _onehot_block(indices_ref, depth_ref, values_ref, out_ref):
    idx = indices_ref[...]
    classes = jax.lax.broadcasted_iota(
        jnp.int32, (_BLOCK_ROWS, _FEATS, _NUM_CLASSES), 2
    )
    off = values_ref[0]
    on = values_ref[1]
    hit = (idx == classes) & (classes < depth_ref[0])
    out_ref[...] = jnp.where(hit, on, off)


def kernel(indices, depth, values):
    grid = _ROWS // _BLOCK_ROWS
    depth_arr = jnp.asarray(depth, dtype=jnp.int32).reshape(1)
    idx3 = indices.reshape(_ROWS, _FEATS, 1)
    return pl.pallas_call(
        _onehot_block,
        grid=(grid,),
        in_specs=[
            pl.BlockSpec((_BLOCK_ROWS, _FEATS, 1), lambda i: (i, 0, 0)),
            pl.BlockSpec(memory_space=pltpu.SMEM),
            pl.BlockSpec(memory_space=pltpu.SMEM),
        ],
        out_specs=pl.BlockSpec(
            (_BLOCK_ROWS, _FEATS, _NUM_CLASSES), lambda i: (i, 0, 0)
        ),
        out_shape=jax.ShapeDtypeStruct(
            (_ROWS, _FEATS, _NUM_CLASSES), jnp.float32
        ),
    )(idx3, depth_arr, values)


# speedup vs baseline: 6.4742x; 4.8617x over previous
import jax
import jax.numpy as jnp
from jax.experimental import pallas as pl
from jax.experimental.pallas import tpu as pltpu

_B = 4096
_F = 26
_C = 1000
_CB = 200


def _onehot_block(idx_ref, depth_ref, values_ref, out_ref):
    c0 = pl.program_id(1) * _CB
    idx = idx_ref[...]
    depth = depth_ref[0]
    idx_eff = jnp.where(idx < depth, idx, -1)
    cls = jax.lax.broadcasted_iota(jnp.int32, (1, _CB, _B), 1) + c0
    out_ref[...] = jnp.where(cls == idx_eff, values_ref[1], values_ref[0])


def kernel(indices, depth, values):
    depth_arr = jnp.asarray(depth, dtype=jnp.int32).reshape(1)
    idx_t = indices.T.reshape(_F, 1, _B)
    out_t = pl.pallas_call(
        _onehot_block,
        grid=(_F, _C // _CB),
        in_specs=[
            pl.BlockSpec((1, 1, _B), lambda f, c: (f, 0, 0)),
            pl.BlockSpec(memory_space=pltpu.SMEM),
            pl.BlockSpec(memory_space=pltpu.SMEM),
        ],
        out_specs=pl.BlockSpec((1, _CB, _B), lambda f, c: (f, c, 0)),
        out_shape=jax.ShapeDtypeStruct((_F, _C, _B), jnp.float32),
    )(idx_t, depth_arr, values)
    return out_t.transpose(2, 0, 1)


# speedup vs baseline: 6.7028x; 1.0353x over previous
import jax
import jax.numpy as jnp
from jax.experimental import pallas as pl
from jax.experimental.pallas import tpu as pltpu

_B = 4096
_F = 26
_C = 1000
_CB = 1000


def _onehot_block(idx_ref, depth_ref, values_ref, out_ref):
    c0 = pl.program_id(1) * _CB
    idx = idx_ref[...]
    depth = depth_ref[0]
    idx_eff = jnp.where(idx < depth, idx, -1)
    cls = jax.lax.broadcasted_iota(jnp.int32, (1, _CB, _B), 1) + c0
    out_ref[...] = jnp.where(cls == idx_eff, values_ref[1], values_ref[0])


def kernel(indices, depth, values):
    depth_arr = jnp.asarray(depth, dtype=jnp.int32).reshape(1)
    idx_t = indices.T.reshape(_F, 1, _B)
    out_t = pl.pallas_call(
        _onehot_block,
        grid=(_F, _C // _CB),
        in_specs=[
            pl.BlockSpec((1, 1, _B), lambda f, c: (f, 0, 0)),
            pl.BlockSpec(memory_space=pltpu.SMEM),
            pl.BlockSpec(memory_space=pltpu.SMEM),
        ],
        out_specs=pl.BlockSpec((1, _CB, _B), lambda f, c: (f, c, 0)),
        out_shape=jax.ShapeDtypeStruct((_F, _C, _B), jnp.float32),
    )(idx_t, depth_arr, values)
    return out_t.transpose(2, 0, 1)


# speedup vs baseline: 6.8089x; 1.0158x over previous
import jax
import jax.numpy as jnp
from jax.experimental import pallas as pl
from jax.experimental.pallas import tpu as pltpu

_B = 4096
_F = 26
_C = 1000
_CB = 1000


def _onehot_block(idx_ref, depth_ref, values_ref, out_ref):
    c0 = pl.program_id(1) * _CB
    idx = idx_ref[pl.ds(pl.program_id(0), 1), :]
    depth = depth_ref[0]
    idx_eff = jnp.where(idx < depth, idx, -1)
    cls = jax.lax.broadcasted_iota(jnp.int32, (_CB, _B), 0) + c0
    out_ref[...] = jnp.where(cls == idx_eff, values_ref[1], values_ref[0])


def kernel(indices, depth, values):
    depth_arr = jnp.asarray(depth, dtype=jnp.int32).reshape(1)
    idx_t = indices.T
    out_t = pl.pallas_call(
        _onehot_block,
        grid=(_F, _C // _CB),
        in_specs=[
            pl.BlockSpec((_F, _B), lambda f, c: (0, 0)),
            pl.BlockSpec(memory_space=pltpu.SMEM),
            pl.BlockSpec(memory_space=pltpu.SMEM),
        ],
        out_specs=pl.BlockSpec((None, _CB, _B), lambda f, c: (f, c, 0)),
        out_shape=jax.ShapeDtypeStruct((_F, _C, _B), jnp.float32),
    )(idx_t, depth_arr, values)
    return out_t.transpose(2, 0, 1)
